# baseline (speedup 1.0000x reference)
import functools

import jax
import jax.numpy as jnp
from jax import lax
from jax.experimental import pallas as pl
from jax.experimental.pallas import tpu as pltpu
from jax.experimental.pallas import tpu_sc as plsc

NC, NS, L = 2, 16, 16
NW = NC * NS
D = 64
SCALE = 8.0
SUB = 128
CHUNK = 512


@functools.partial(jax.jit, static_argnames=("B",))
def _sc_lookup(idx_flat, table, B):
    b_per_w = B // NW
    n_chunks = b_per_w // CHUNK
    mesh = plsc.VectorSubcoreMesh(core_axis_name="c", subcore_axis_name="s")

    @functools.partial(
        pl.kernel,
        out_type=jax.ShapeDtypeStruct((B, D), jnp.float32),
        mesh=mesh,
        scratch_types=[
            pltpu.VMEM((CHUNK,), jnp.int32),
            pltpu.VMEM((CHUNK, D), jnp.float32),
            pltpu.SemaphoreType.DMA,
        ],
        compiler_params=pltpu.CompilerParams(use_tc_tiling_on_sc=False),
    )
    def k(idx_hbm, table_hbm, out_hbm, idx_v, rows_v, sem):
        wid = lax.axis_index("s") * NC + lax.axis_index("c")
        base = wid * b_per_w

        def chunk_body(g, carry):
            off = base + g * CHUNK
            pltpu.sync_copy(idx_hbm.at[pl.ds(off, CHUNK)], idx_v)
            descs = [
                pltpu.async_copy(
                    table_hbm.at[idx_v.at[pl.ds(j * SUB, SUB)]],
                    rows_v.at[pl.ds(j * SUB, SUB)],
                    sem,
                )
                for j in range(CHUNK // SUB)
            ]
            for dsc in descs:
                dsc.wait()

            def grp_body(gg, carry):
                g16 = idx_v[pl.ds(gg * L, L)]
                m16 = jnp.where(g16 != 0, SCALE, 0.0).astype(jnp.float32)
                for r in range(L):
                    m = m16.at[jnp.full((L,), r, jnp.int32)].get(
                        mode="promise_in_bounds")
                    row = gg * L + r
                    for kk in range(D // L):
                        v = rows_v[row, pl.ds(kk * L, L)]
                        rows_v[row, pl.ds(kk * L, L)] = v * m
                return carry

            lax.fori_loop(0, CHUNK // L, grp_body, 0, unroll=False)
            pltpu.sync_copy(rows_v, out_hbm.at[pl.ds(off, CHUNK)])
            return carry

        lax.fori_loop(0, n_chunks, chunk_body, 0, unroll=False)

    return k(idx_flat, table)


def kernel(inputs, shared_weights):
    B = inputs.size
    idx_flat = inputs.reshape(B).astype(jnp.int32)
    out = _sc_lookup(idx_flat, shared_weights, B)
    return out.reshape(inputs.shape + (D,))


# speedup vs baseline: 1.2856x; 1.2856x over previous
import functools

import jax
import jax.numpy as jnp
from jax import lax
from jax.experimental import pallas as pl
from jax.experimental.pallas import tpu as pltpu
from jax.experimental.pallas import tpu_sc as plsc

NC, NS, L = 2, 16, 16
NW = NC * NS
D = 64
SCALE = 8.0
SUB = 128
CHUNK = 256
NBUF = 4


@functools.partial(jax.jit, static_argnames=("B",))
def _sc_lookup(idx_flat, table, B):
    b_per_w = B // NW
    n_chunks = b_per_w // CHUNK
    assert n_chunks % NBUF == 0 and n_chunks >= 2 * NBUF
    mesh = plsc.VectorSubcoreMesh(core_axis_name="c", subcore_axis_name="s")

    @functools.partial(
        pl.kernel,
        out_type=jax.ShapeDtypeStruct((B, D), jnp.float32),
        mesh=mesh,
        scratch_types=[
            pltpu.VMEM((NBUF, CHUNK), jnp.int32),
            pltpu.VMEM((NBUF, CHUNK, D), jnp.float32),
            pltpu.SemaphoreType.DMA((NBUF,)),
            pltpu.SemaphoreType.DMA((NBUF,)),
        ],
        compiler_params=pltpu.CompilerParams(use_tc_tiling_on_sc=False),
    )
    def k(idx_hbm, table_hbm, out_hbm, idx_v, rows_v, sem_g, sem_o):
        wid = lax.axis_index("s") * NC + lax.axis_index("c")
        base = wid * b_per_w

        def stage_and_fire(c, b):
            off = base + c * CHUNK
            pltpu.sync_copy(idx_hbm.at[pl.ds(off, CHUNK)], idx_v.at[b])
            for j in range(CHUNK // SUB):
                pltpu.async_copy(
                    table_hbm.at[idx_v.at[b, pl.ds(j * SUB, SUB)]],
                    rows_v.at[b, pl.ds(j * SUB, SUB)],
                    sem_g.at[b],
                )

        def drain_gathers(b):
            for j in range(CHUNK // SUB):
                pltpu.make_async_copy(
                    table_hbm.at[idx_v.at[b, pl.ds(j * SUB, SUB)]],
                    rows_v.at[b, pl.ds(j * SUB, SUB)],
                    sem_g.at[b],
                ).wait()

        def wait_outcopy(b):
            pltpu.make_async_copy(
                rows_v.at[b], out_hbm.at[pl.ds(0, CHUNK)], sem_o.at[b]
            ).wait()

        def compute(b):
            def grp_body(gg, carry):
                g16 = idx_v[b, pl.ds(gg * L, L)]
                m16 = jnp.where(g16 != 0, SCALE, 0.0).astype(jnp.float32)
                for r in range(L):
                    m = m16.at[jnp.full((L,), r, jnp.int32)].get(
                        mode="promise_in_bounds")
                    row = gg * L + r
                    for kk in range(D // L):
                        v = rows_v[b, row, pl.ds(kk * L, L)]
                        rows_v[b, row, pl.ds(kk * L, L)] = v * m
                return carry

            lax.fori_loop(0, CHUNK // L, grp_body, 0, unroll=False)

        for c in range(NBUF - 1):
            stage_and_fire(c, c)

        def outer_body(g, carry):
            for b in range(NBUF):
                c = g * NBUF + b
                drain_gathers(b)
                compute(b)
                pltpu.async_copy(
                    rows_v.at[b],
                    out_hbm.at[pl.ds(base + c * CHUNK, CHUNK)],
                    sem_o.at[b],
                )
                bp = (b + NBUF - 1) % NBUF

                @pl.when(c + NBUF - 1 < n_chunks)
                def _():
                    @pl.when(c >= 1)
                    def _():
                        wait_outcopy(bp)

                    stage_and_fire(c + NBUF - 1, bp)

            return carry

        lax.fori_loop(0, n_chunks // NBUF, outer_body, 0, unroll=False)

        for c in range(n_chunks - NBUF, n_chunks):
            wait_outcopy(c % NBUF)

    return k(idx_flat, table)


def kernel(inputs, shared_weights):
    B = inputs.size
    idx_flat = inputs.reshape(B).astype(jnp.int32)
    out = _sc_lookup(idx_flat, shared_weights, B)
    return out.reshape(inputs.shape + (D,))


# speedup vs baseline: 1.3383x; 1.0409x over previous
import functools

import jax
import jax.numpy as jnp
from jax import lax
from jax.experimental import pallas as pl
from jax.experimental.pallas import tpu as pltpu
from jax.experimental.pallas import tpu_sc as plsc

NC, NS, L = 2, 16, 16
NW = NC * NS
D = 64
SCALE = 8.0
SUB = 128
CHUNK = 256
NBUF = 4


@functools.partial(jax.jit, static_argnames=("B",))
def _sc_lookup(idx_flat, table, B):
    b_per_w = B // NW
    n_chunks = b_per_w // CHUNK
    assert n_chunks % NBUF == 0 and n_chunks >= 2 * NBUF
    mesh = plsc.VectorSubcoreMesh(core_axis_name="c", subcore_axis_name="s")

    @functools.partial(
        pl.kernel,
        out_type=jax.ShapeDtypeStruct((B, D), jnp.float32),
        mesh=mesh,
        scratch_types=[
            pltpu.VMEM((b_per_w,), jnp.int32),
            pltpu.VMEM((NBUF, CHUNK, D), jnp.float32),
            pltpu.SemaphoreType.DMA((NBUF,)),
            pltpu.SemaphoreType.DMA((NBUF,)),
        ],
        compiler_params=pltpu.CompilerParams(use_tc_tiling_on_sc=False),
    )
    def k(idx_hbm, table_hbm, out_hbm, idx_v, rows_v, sem_g, sem_o):
        wid = lax.axis_index("s") * NC + lax.axis_index("c")
        base = wid * b_per_w

        pltpu.sync_copy(idx_hbm.at[pl.ds(base, b_per_w)], idx_v)

        def stage_and_fire(c, b):
            for j in range(CHUNK // SUB):
                pltpu.async_copy(
                    table_hbm.at[idx_v.at[pl.ds(c * CHUNK + j * SUB, SUB)]],
                    rows_v.at[b, pl.ds(j * SUB, SUB)],
                    sem_g.at[b],
                )

        def drain_gathers(c, b):
            for j in range(CHUNK // SUB):
                pltpu.make_async_copy(
                    table_hbm.at[idx_v.at[pl.ds(c * CHUNK + j * SUB, SUB)]],
                    rows_v.at[b, pl.ds(j * SUB, SUB)],
                    sem_g.at[b],
                ).wait()

        def wait_outcopy(b):
            pltpu.make_async_copy(
                rows_v.at[b], out_hbm.at[pl.ds(0, CHUNK)], sem_o.at[b]
            ).wait()

        def compute(c, b):
            def grp_body(gg, carry):
                g16 = idx_v[pl.ds(c * CHUNK + gg * L, L)]
                m16 = jnp.where(g16 != 0, SCALE, 0.0).astype(jnp.float32)
                for r in range(L):
                    m = m16.at[jnp.full((L,), r, jnp.int32)].get(
                        mode="promise_in_bounds")
                    row = gg * L + r
                    for kk in range(D // L):
                        v = rows_v[b, row, pl.ds(kk * L, L)]
                        rows_v[b, row, pl.ds(kk * L, L)] = v * m
                return carry

            lax.fori_loop(0, CHUNK // L, grp_body, 0, unroll=False)

        for c in range(NBUF - 1):
            stage_and_fire(c, c)

        def outer_body(g, carry):
            for b in range(NBUF):
                c = g * NBUF + b
                drain_gathers(c, b)
                compute(c, b)
                pltpu.async_copy(
                    rows_v.at[b],
                    out_hbm.at[pl.ds(base + c * CHUNK, CHUNK)],
                    sem_o.at[b],
                )
                bp = (b + NBUF - 1) % NBUF

                @pl.when(c + NBUF - 1 < n_chunks)
                def _():
                    @pl.when(c >= 1)
                    def _():
                        wait_outcopy(bp)

                    stage_and_fire(c + NBUF - 1, bp)

            return carry

        lax.fori_loop(0, n_chunks // NBUF, outer_body, 0, unroll=False)

        for c in range(n_chunks - NBUF, n_chunks):
            wait_outcopy(c % NBUF)

    return k(idx_flat, table)


def kernel(inputs, shared_weights):
    B = inputs.size
    idx_flat = inputs.reshape(B).astype(jnp.int32)
    out = _sc_lookup(idx_flat, shared_weights, B)
    return out.reshape(inputs.shape + (D,))
